# baseline (speedup 1.0000x reference)
import functools
import jax
import jax.numpy as jnp
from jax import lax
from jax.experimental import pallas as pl
from jax.experimental.pallas import tpu as pltpu
from jax.experimental.pallas import tpu_sc as plsc

N = 10000
E = 320000
D = 128
OUT = 64

CHUNK = 128
N_PAD = 10112
ROWS_PER_TILE = N_PAD // 16
N_CHUNKS = -(-E // (CHUNK * 64)) * 64
E_PAD = N_CHUNKS * CHUNK
CHUNKS_PER_TILE = N_CHUNKS // 32
CHUNKS_PER_CORE = N_CHUNKS // 2
HALF = CHUNKS_PER_TILE // 2
PAIRS_H = HALF // 2

_mesh = functools.partial(
    plsc.VectorSubcoreMesh, core_axis_name="c", subcore_axis_name="s")


def _deg_kernel(dst2_hbm, ones_hbm, zeros_hbm, out_hbm,
                idx_d, ones_v, acc, sem):
    c = lax.axis_index("c")
    s = lax.axis_index("s")
    rbase = pl.multiple_of(s * ROWS_PER_TILE, 8)
    cbase = c * CHUNKS_PER_CORE + s * CHUNKS_PER_TILE
    pltpu.sync_copy(ones_hbm, ones_v)
    pltpu.sync_copy(zeros_hbm.at[pl.ds(rbase, ROWS_PER_TILE)],
                    acc.at[pl.ds(rbase, ROWS_PER_TILE)])
    plsc.subcore_barrier()

    def body(j, carry):
        pltpu.sync_copy(dst2_hbm.at[cbase + j], idx_d)
        pltpu.sync_copy(ones_v, acc.at[idx_d], add=True)
        return carry

    lax.fori_loop(0, CHUNKS_PER_TILE, body, 0)
    plsc.subcore_barrier()

    obase = pl.multiple_of(c * N_PAD + rbase, 8)
    pltpu.sync_copy(acc.at[pl.ds(rbase, ROWS_PER_TILE)],
                    out_hbm.at[pl.ds(obase, ROWS_PER_TILE)])


def _spmm_kernel(y_hbm, src2_hbm, dst2_hbm, zeros_hbm, out_hbm,
                 idx_s, idx_d, rows_v, acc, sem):
    c = lax.axis_index("c")
    s = lax.axis_index("s")
    rbase = pl.multiple_of(s * ROWS_PER_TILE, 8)
    cbase = c * CHUNKS_PER_CORE + s * CHUNKS_PER_TILE
    pltpu.sync_copy(zeros_hbm.at[pl.ds(rbase, ROWS_PER_TILE)],
                    acc.at[pl.ds(rbase, ROWS_PER_TILE)])
    plsc.subcore_barrier()

    def body(j, carry):
        pltpu.sync_copy(src2_hbm.at[cbase + j], idx_s)
        pltpu.sync_copy(dst2_hbm.at[cbase + j], idx_d)
        pltpu.async_copy(y_hbm.at[idx_s], rows_v, sem).wait()
        pltpu.sync_copy(rows_v, acc.at[idx_d], add=True)
        return carry

    lax.fori_loop(0, CHUNKS_PER_TILE, body, 0)
    plsc.subcore_barrier()

    obase = pl.multiple_of(c * N_PAD + rbase, 8)
    pltpu.sync_copy(acc.at[pl.ds(rbase, ROWS_PER_TILE)],
                    out_hbm.at[pl.ds(obase, ROWS_PER_TILE)])


def _deg_call(dst_pad):
    ones = jnp.ones((CHUNK, D), jnp.float32)
    zeros = jnp.zeros((N_PAD, D), jnp.float32)
    k = pl.kernel(
        _deg_kernel,
        mesh=_mesh(),
        out_type=jax.ShapeDtypeStruct((2 * N_PAD, D), jnp.float32),
        scratch_types=[
            pltpu.VMEM((CHUNK,), jnp.int32),
            pltpu.VMEM((CHUNK, D), jnp.float32),
            pltpu.VMEM_SHARED((N_PAD, D), jnp.float32),
            pltpu.SemaphoreType.DMA,
        ],
    )
    return k(dst_pad, ones, zeros)


def _spmm_call(y, src_pad, dst_pad):
    zeros = jnp.zeros((N_PAD, D), jnp.float32)
    k = pl.kernel(
        _spmm_kernel,
        mesh=_mesh(),
        out_type=jax.ShapeDtypeStruct((2 * N_PAD, D), jnp.float32),
        scratch_types=[
            pltpu.VMEM((CHUNK,), jnp.int32),
            pltpu.VMEM((CHUNK,), jnp.int32),
            pltpu.VMEM((CHUNK, D), jnp.float32),
            pltpu.VMEM_SHARED((N_PAD, D), jnp.float32),
            pltpu.SemaphoreType.DMA,
        ],
    )
    return k(y, src_pad, dst_pad, zeros)


RB = 1000
GRID = N // RB


def _scale_kernel(d0_ref, d1_ref, x_ref, y_ref, dinv_ref):
    deg = d0_ref[:, 0:1] + d1_ref[:, 0:1] + 1.0
    dinv = lax.rsqrt(jnp.maximum(deg, 1.0))
    y_ref[...] = x_ref[...] * dinv
    dinv_ref[...] = jnp.broadcast_to(dinv, dinv_ref.shape)


def _tc_scale(degp0, degp1, x):
    return pl.pallas_call(
        _scale_kernel,
        grid=(GRID,),
        in_specs=[
            pl.BlockSpec((RB, D), lambda i: (i, 0)),
            pl.BlockSpec((RB, D), lambda i: (i, 0)),
            pl.BlockSpec((RB, D), lambda i: (i, 0)),
        ],
        out_specs=[
            pl.BlockSpec((RB, D), lambda i: (i, 0)),
            pl.BlockSpec((RB, 16), lambda i: (i, 0)),
        ],
        out_shape=[jax.ShapeDtypeStruct((N, D), jnp.float32),
                   jax.ShapeDtypeStruct((N, 16), jnp.float32)],
    )(degp0, degp1, x)


def _layer_kernel(p0_ref, p1_ref, x_ref, dinv_ref, w_ref, b_ref,
                  h_ref, y_ref):
    dinv = dinv_ref[:, 0:1]
    agg = dinv * (p0_ref[...] + p1_ref[...]) + (dinv * dinv) * x_ref[...]
    h = jnp.dot(agg, w_ref[...], preferred_element_type=jnp.float32)
    h = jax.nn.sigmoid(h + b_ref[...])
    h_ref[...] = h
    y_ref[...] = h * dinv


def _tc_layer(p0, p1, x, dinv, w, b):
    return pl.pallas_call(
        _layer_kernel,
        grid=(GRID,),
        in_specs=[
            pl.BlockSpec((RB, D), lambda i: (i, 0)),
            pl.BlockSpec((RB, D), lambda i: (i, 0)),
            pl.BlockSpec((RB, D), lambda i: (i, 0)),
            pl.BlockSpec((RB, 16), lambda i: (i, 0)),
            pl.BlockSpec((D, D), lambda i: (0, 0)),
            pl.BlockSpec((1, D), lambda i: (0, 0)),
        ],
        out_specs=[
            pl.BlockSpec((RB, D), lambda i: (i, 0)),
            pl.BlockSpec((RB, D), lambda i: (i, 0)),
        ],
        out_shape=[jax.ShapeDtypeStruct((N, D), jnp.float32),
                   jax.ShapeDtypeStruct((N, D), jnp.float32)],
    )(p0, p1, x, dinv, w, b)


def _final_kernel(p0_ref, p1_ref, x_ref, dinv_ref, w_ref, b_ref,
                  wm1_ref, bm1_ref, wm2_ref, bm2_ref, o_ref):
    dinv = dinv_ref[:, 0:1]
    agg = dinv * (p0_ref[...] + p1_ref[...]) + (dinv * dinv) * x_ref[...]
    h = jnp.dot(agg, w_ref[...], preferred_element_type=jnp.float32)
    h = jax.nn.sigmoid(h + b_ref[...])
    m = jnp.dot(h, wm1_ref[...], preferred_element_type=jnp.float32)
    m = jnp.maximum(m + bm1_ref[...], 0.0)
    o = jnp.dot(m, wm2_ref[...], preferred_element_type=jnp.float32)
    o_ref[...] = o + bm2_ref[...]


def _tc_final(p0, p1, h1, dinv, w2, b2, wm1, bm1, wm2, bm2):
    return pl.pallas_call(
        _final_kernel,
        grid=(GRID,),
        in_specs=[
            pl.BlockSpec((RB, D), lambda i: (i, 0)),
            pl.BlockSpec((RB, D), lambda i: (i, 0)),
            pl.BlockSpec((RB, D), lambda i: (i, 0)),
            pl.BlockSpec((RB, 16), lambda i: (i, 0)),
            pl.BlockSpec((D, D), lambda i: (0, 0)),
            pl.BlockSpec((1, D), lambda i: (0, 0)),
            pl.BlockSpec((D, D), lambda i: (0, 0)),
            pl.BlockSpec((1, D), lambda i: (0, 0)),
            pl.BlockSpec((D, OUT), lambda i: (0, 0)),
            pl.BlockSpec((1, OUT), lambda i: (0, 0)),
        ],
        out_specs=pl.BlockSpec((RB, OUT), lambda i: (i, 0)),
        out_shape=jax.ShapeDtypeStruct((N, OUT), jnp.float32),
    )(p0, p1, h1, dinv, w2, b2, wm1, bm1, wm2, bm2)


def kernel(features, edge_index, W1, b1, W2, b2, Wm1, bm1, Wm2, bm2):
    npad = E_PAD - E
    src_pad = jnp.concatenate(
        [edge_index[0], jnp.zeros((npad,), jnp.int32)]
    ).reshape(N_CHUNKS, CHUNK)
    dst_pad = jnp.concatenate(
        [edge_index[1], jnp.full((npad,), N, jnp.int32)]
    ).reshape(N_CHUNKS, CHUNK)

    degp = _deg_call(dst_pad)
    y1, dinv = _tc_scale(degp[:N], degp[N_PAD:N_PAD + N], features)

    p = _spmm_call(y1, src_pad, dst_pad)
    h1, y2 = _tc_layer(p[:N], p[N_PAD:N_PAD + N], features, dinv,
                       W1, b1.reshape(1, D))

    q = _spmm_call(y2, src_pad, dst_pad)
    out = _tc_final(q[:N], q[N_PAD:N_PAD + N], h1, dinv, W2, b2.reshape(1, D),
                    Wm1, bm1.reshape(1, D), Wm2, bm2.reshape(1, OUT))
    return out


# speedup vs baseline: 1.3633x; 1.3633x over previous
import functools
import jax
import jax.numpy as jnp
from jax import lax
from jax.experimental import pallas as pl
from jax.experimental.pallas import tpu as pltpu
from jax.experimental.pallas import tpu_sc as plsc

N = 10000
E = 320000
D = 128
OUT = 64

CHUNK = 128
N_PAD = 10112
ROWS_PER_TILE = N_PAD // 16
N_CHUNKS = -(-E // (CHUNK * 32)) * 32
E_PAD = N_CHUNKS * CHUNK
CHUNKS_PER_TILE = N_CHUNKS // 32
CHUNKS_PER_CORE = N_CHUNKS // 2

_mesh = functools.partial(
    plsc.VectorSubcoreMesh, core_axis_name="c", subcore_axis_name="s")


def _deg_kernel(dst_hbm, ones_hbm, zeros_hbm, out_hbm,
                idx_d, ones_v, acc, sem):
    c = lax.axis_index("c")
    s = lax.axis_index("s")
    rbase = pl.multiple_of(s * ROWS_PER_TILE, 8)
    cbase = c * CHUNKS_PER_CORE + s * CHUNKS_PER_TILE
    pltpu.sync_copy(ones_hbm, ones_v)
    pltpu.sync_copy(zeros_hbm.at[pl.ds(rbase, ROWS_PER_TILE)],
                    acc.at[pl.ds(rbase, ROWS_PER_TILE)])
    plsc.subcore_barrier()

    def body(j, carry):
        off = pl.multiple_of((cbase + j) * CHUNK, CHUNK)
        pltpu.sync_copy(dst_hbm.at[pl.ds(off, CHUNK)], idx_d)
        pltpu.sync_copy(ones_v, acc.at[idx_d], add=True)
        return carry

    lax.fori_loop(0, CHUNKS_PER_TILE, body, 0)
    plsc.subcore_barrier()

    obase = pl.multiple_of(c * N_PAD + rbase, 8)
    pltpu.sync_copy(acc.at[pl.ds(rbase, ROWS_PER_TILE)],
                    out_hbm.at[pl.ds(obase, ROWS_PER_TILE)])


def _spmm_kernel(y_hbm, src_hbm, dst_hbm, zeros_hbm, out_hbm,
                 idx_s, idx_d, rows_v, acc, sem):
    c = lax.axis_index("c")
    s = lax.axis_index("s")
    rbase = pl.multiple_of(s * ROWS_PER_TILE, 8)
    cbase = c * CHUNKS_PER_CORE + s * CHUNKS_PER_TILE
    pltpu.sync_copy(zeros_hbm.at[pl.ds(rbase, ROWS_PER_TILE)],
                    acc.at[pl.ds(rbase, ROWS_PER_TILE)])
    plsc.subcore_barrier()

    def body(j, carry):
        off = pl.multiple_of((cbase + j) * CHUNK, CHUNK)
        pltpu.sync_copy(src_hbm.at[pl.ds(off, CHUNK)], idx_s)
        pltpu.sync_copy(dst_hbm.at[pl.ds(off, CHUNK)], idx_d)
        pltpu.async_copy(y_hbm.at[idx_s], rows_v, sem).wait()
        pltpu.sync_copy(rows_v, acc.at[idx_d], add=True)
        return carry

    lax.fori_loop(0, CHUNKS_PER_TILE, body, 0)
    plsc.subcore_barrier()

    obase = pl.multiple_of(c * N_PAD + rbase, 8)
    pltpu.sync_copy(acc.at[pl.ds(rbase, ROWS_PER_TILE)],
                    out_hbm.at[pl.ds(obase, ROWS_PER_TILE)])


def _deg_call(dst_pad):
    ones = jnp.ones((CHUNK, D), jnp.float32)
    zeros = jnp.zeros((N_PAD, D), jnp.float32)
    k = pl.kernel(
        _deg_kernel,
        mesh=_mesh(),
        out_type=jax.ShapeDtypeStruct((2 * N_PAD, D), jnp.float32),
        scratch_types=[
            pltpu.VMEM((CHUNK,), jnp.int32),
            pltpu.VMEM((CHUNK, D), jnp.float32),
            pltpu.VMEM_SHARED((N_PAD, D), jnp.float32),
            pltpu.SemaphoreType.DMA,
        ],
    )
    return k(dst_pad, ones, zeros)


def _spmm_call(y, src_pad, dst_pad):
    zeros = jnp.zeros((N_PAD, D), jnp.float32)
    k = pl.kernel(
        _spmm_kernel,
        mesh=_mesh(),
        out_type=jax.ShapeDtypeStruct((2 * N_PAD, D), jnp.float32),
        scratch_types=[
            pltpu.VMEM((CHUNK,), jnp.int32),
            pltpu.VMEM((CHUNK,), jnp.int32),
            pltpu.VMEM((CHUNK, D), jnp.float32),
            pltpu.VMEM_SHARED((N_PAD, D), jnp.float32),
            pltpu.SemaphoreType.DMA,
        ],
    )
    return k(y, src_pad, dst_pad, zeros)


RB = 1000
GRID = N // RB


def _scale_kernel(d0_ref, d1_ref, x_ref, y_ref, dinv_ref):
    deg = d0_ref[:, 0:1] + d1_ref[:, 0:1] + 1.0
    dinv = lax.rsqrt(jnp.maximum(deg, 1.0))
    y_ref[...] = x_ref[...] * dinv
    dinv_ref[...] = jnp.broadcast_to(dinv, dinv_ref.shape)


def _tc_scale(degp0, degp1, x):
    return pl.pallas_call(
        _scale_kernel,
        grid=(GRID,),
        in_specs=[
            pl.BlockSpec((RB, D), lambda i: (i, 0)),
            pl.BlockSpec((RB, D), lambda i: (i, 0)),
            pl.BlockSpec((RB, D), lambda i: (i, 0)),
        ],
        out_specs=[
            pl.BlockSpec((RB, D), lambda i: (i, 0)),
            pl.BlockSpec((RB, 16), lambda i: (i, 0)),
        ],
        out_shape=[jax.ShapeDtypeStruct((N, D), jnp.float32),
                   jax.ShapeDtypeStruct((N, 16), jnp.float32)],
    )(degp0, degp1, x)


def _layer_kernel(p0_ref, p1_ref, x_ref, dinv_ref, w_ref, b_ref,
                  h_ref, y_ref):
    dinv = dinv_ref[:, 0:1]
    agg = dinv * (p0_ref[...] + p1_ref[...]) + (dinv * dinv) * x_ref[...]
    h = jnp.dot(agg, w_ref[...], preferred_element_type=jnp.float32)
    h = jax.nn.sigmoid(h + b_ref[...])
    h_ref[...] = h
    y_ref[...] = h * dinv


def _tc_layer(p0, p1, x, dinv, w, b):
    return pl.pallas_call(
        _layer_kernel,
        grid=(GRID,),
        in_specs=[
            pl.BlockSpec((RB, D), lambda i: (i, 0)),
            pl.BlockSpec((RB, D), lambda i: (i, 0)),
            pl.BlockSpec((RB, D), lambda i: (i, 0)),
            pl.BlockSpec((RB, 16), lambda i: (i, 0)),
            pl.BlockSpec((D, D), lambda i: (0, 0)),
            pl.BlockSpec((1, D), lambda i: (0, 0)),
        ],
        out_specs=[
            pl.BlockSpec((RB, D), lambda i: (i, 0)),
            pl.BlockSpec((RB, D), lambda i: (i, 0)),
        ],
        out_shape=[jax.ShapeDtypeStruct((N, D), jnp.float32),
                   jax.ShapeDtypeStruct((N, D), jnp.float32)],
    )(p0, p1, x, dinv, w, b)


def _final_kernel(p0_ref, p1_ref, x_ref, dinv_ref, w_ref, b_ref,
                  wm1_ref, bm1_ref, wm2_ref, bm2_ref, o_ref):
    dinv = dinv_ref[:, 0:1]
    agg = dinv * (p0_ref[...] + p1_ref[...]) + (dinv * dinv) * x_ref[...]
    h = jnp.dot(agg, w_ref[...], preferred_element_type=jnp.float32)
    h = jax.nn.sigmoid(h + b_ref[...])
    m = jnp.dot(h, wm1_ref[...], preferred_element_type=jnp.float32)
    m = jnp.maximum(m + bm1_ref[...], 0.0)
    o = jnp.dot(m, wm2_ref[...], preferred_element_type=jnp.float32)
    o_ref[...] = o + bm2_ref[...]


def _tc_final(p0, p1, h1, dinv, w2, b2, wm1, bm1, wm2, bm2):
    return pl.pallas_call(
        _final_kernel,
        grid=(GRID,),
        in_specs=[
            pl.BlockSpec((RB, D), lambda i: (i, 0)),
            pl.BlockSpec((RB, D), lambda i: (i, 0)),
            pl.BlockSpec((RB, D), lambda i: (i, 0)),
            pl.BlockSpec((RB, 16), lambda i: (i, 0)),
            pl.BlockSpec((D, D), lambda i: (0, 0)),
            pl.BlockSpec((1, D), lambda i: (0, 0)),
            pl.BlockSpec((D, D), lambda i: (0, 0)),
            pl.BlockSpec((1, D), lambda i: (0, 0)),
            pl.BlockSpec((D, OUT), lambda i: (0, 0)),
            pl.BlockSpec((1, OUT), lambda i: (0, 0)),
        ],
        out_specs=pl.BlockSpec((RB, OUT), lambda i: (i, 0)),
        out_shape=jax.ShapeDtypeStruct((N, OUT), jnp.float32),
    )(p0, p1, h1, dinv, w2, b2, wm1, bm1, wm2, bm2)


def kernel(features, edge_index, W1, b1, W2, b2, Wm1, bm1, Wm2, bm2):
    npad = E_PAD - E
    src_pad = jnp.concatenate(
        [edge_index[0], jnp.zeros((npad,), jnp.int32)])
    dst_pad = jnp.concatenate(
        [edge_index[1], jnp.full((npad,), N, jnp.int32)])

    degp = _deg_call(dst_pad)
    y1, dinv = _tc_scale(degp[:N], degp[N_PAD:N_PAD + N], features)

    p = _spmm_call(y1, src_pad, dst_pad)
    h1, y2 = _tc_layer(p[:N], p[N_PAD:N_PAD + N], features, dinv,
                       W1, b1.reshape(1, D))

    q = _spmm_call(y2, src_pad, dst_pad)
    out = _tc_final(q[:N], q[N_PAD:N_PAD + N], h1, dinv, W2, b2.reshape(1, D),
                    Wm1, bm1.reshape(1, D), Wm2, bm2.reshape(1, OUT))
    return out
